# baseline (speedup 1.0000x reference)
import functools
import math

import jax
import jax.numpy as jnp
from jax import lax
from jax.experimental import pallas as pl
from jax.experimental.pallas import tpu as pltpu
from jax.experimental.pallas import tpu_sc as plsc

VOCAB = 100000
D = 768
B = 4
S = 4096
N = B * S
SCALE = math.sqrt(float(D))

_info = plsc.get_sparse_core_info()
NC = _info.num_cores
NS = _info.num_subcores
NW = NC * NS
S_W = S // NW
R = 16
NCH = S_W // R
LANES = 16
JV = D // LANES


def _sc_embed(idx_arr, table, pe_s):
    mesh = plsc.VectorSubcoreMesh(core_axis_name="c", subcore_axis_name="s")

    @functools.partial(
        pl.kernel,
        mesh=mesh,
        out_type=jax.ShapeDtypeStruct((N, D), jnp.float32),
        scratch_types=[
            pltpu.VMEM((NCH * B, R), jnp.int32),
            pltpu.VMEM((2 * B, R, D), jnp.float32),
            pltpu.VMEM((2, R, D), jnp.float32),
            pltpu.SemaphoreType.DMA,
            pltpu.SemaphoreType.DMA,
            pltpu.SemaphoreType.DMA,
            pltpu.SemaphoreType.DMA,
            pltpu.SemaphoreType.DMA,
            pltpu.SemaphoreType.DMA,
        ],
    )
    def k(idx_hbm, table_hbm, pe_hbm, out_hbm,
          idx_v, rows_v, pe_v, g0, g1, o0, o1, p0, p1):
        wid = lax.axis_index("s") * NC + lax.axis_index("c")
        sbase = wid * S_W
        g_sem = (g0, g1)
        o_sem = (o0, o1)
        p_sem = (p0, p1)

        def fire_pe(sc, par):
            pltpu.async_copy(
                pe_hbm.at[pl.ds(sbase + sc * R, R)], pe_v.at[par], p_sem[par])

        def fire_gathers(sc, par):
            for b in range(B):
                pltpu.async_copy(
                    table_hbm.at[idx_v.at[sc * B + b]],
                    rows_v.at[par * B + b], g_sem[par])

        def drain_gathers(sc, par):
            for b in range(B):
                pltpu.make_async_copy(
                    table_hbm.at[idx_v.at[sc * B + b]],
                    rows_v.at[par * B + b], g_sem[par]).wait()

        def drain_outs(par):
            for b in range(B):
                pltpu.make_async_copy(
                    rows_v.at[par * B + b],
                    out_hbm.at[pl.ds(0, R)], o_sem[par]).wait()

        pltpu.sync_copy(idx_hbm.at[wid], idx_v)
        fire_pe(0, 0)
        fire_gathers(0, 0)

        def stage(sc, par, do_prefetch, do_out_drain):
            nxt = 1 - par

            def prefetch():
                fire_pe(sc + 1, nxt)

                def guard():
                    drain_outs(nxt)
                pl.when(do_out_drain)(guard)
                fire_gathers(sc + 1, nxt)
            pl.when(do_prefetch)(prefetch)

            drain_gathers(sc, par)
            pltpu.make_async_copy(
                pe_hbm.at[pl.ds(sbase + sc * R, R)],
                pe_v.at[par], p_sem[par]).wait()

            def row(r, _):
                for j in range(JV):
                    sl = pl.ds(j * LANES, LANES)
                    pe_reg = pe_v[par, r, sl]
                    for b in range(B):
                        slot = par * B + b
                        rows_v[slot, r, sl] = rows_v[slot, r, sl] * SCALE + pe_reg
                return 0

            lax.fori_loop(0, R, row, 0)
            for b in range(B):
                pltpu.async_copy(
                    rows_v.at[par * B + b],
                    out_hbm.at[pl.ds(b * S + sbase + sc * R, R)], o_sem[par])

        def group(g, _):
            stage(2 * g, 0, jnp.bool_(True), g >= 1)
            stage(2 * g + 1, 1, g < (NCH // 2 - 1), jnp.bool_(True))
            return 0

        lax.fori_loop(0, NCH // 2, group, 0)
        drain_outs(0)
        drain_outs(1)

    return k(idx_arr, table, pe_s)


def kernel(x, table, pe):
    idx_arr = (x.reshape(B, NW, NCH, R)
                .transpose(1, 2, 0, 3)
                .reshape(NW, NCH * B, R))
    out = _sc_embed(idx_arr, table, pe[:S])
    return out.reshape(B, S, D)


# speedup vs baseline: 1.2343x; 1.2343x over previous
import functools
import math

import jax
import jax.numpy as jnp
from jax import lax
from jax.experimental import pallas as pl
from jax.experimental.pallas import tpu as pltpu
from jax.experimental.pallas import tpu_sc as plsc

VOCAB = 100000
D = 768
B = 4
S = 4096
N = B * S
SCALE = math.sqrt(float(D))

_info = plsc.get_sparse_core_info()
NC = _info.num_cores
NS = _info.num_subcores
NW = NC * NS
S_W = S // NW
R = 32
NCH = S_W // R
LANES = 16
JV = D // LANES


def _sc_embed(idx_arr, table, pe_s):
    mesh = plsc.VectorSubcoreMesh(core_axis_name="c", subcore_axis_name="s")

    @functools.partial(
        pl.kernel,
        mesh=mesh,
        out_type=jax.ShapeDtypeStruct((N, D), jnp.float32),
        scratch_types=[
            pltpu.VMEM((NCH * B, R), jnp.int32),
            pltpu.VMEM((R, D), jnp.float32),
            pltpu.VMEM((R, D), jnp.float32),
            pltpu.SemaphoreType.DMA,
        ],
    )
    def k(idx_hbm, table_hbm, pe_hbm, out_hbm, idx_v, rows_v, pe_v, sem):
        wid = lax.axis_index("s") * NC + lax.axis_index("c")
        sbase = wid * S_W
        pltpu.sync_copy(idx_hbm.at[wid], idx_v)

        def chunk(sc, _):
            pltpu.sync_copy(pe_hbm.at[pl.ds(sbase + sc * R, R)], pe_v)
            for b in range(B):
                pltpu.async_copy(
                    table_hbm.at[idx_v.at[sc * B + b]], rows_v, sem).wait()

                def row(r, _):
                    for j in range(JV):
                        sl = pl.ds(j * LANES, LANES)
                        rows_v[r, sl] = rows_v[r, sl] * SCALE + pe_v[r, sl]
                    return 0

                lax.fori_loop(0, R, row, 0)
                pltpu.sync_copy(
                    rows_v, out_hbm.at[pl.ds(b * S + sbase + sc * R, R)])
            return 0

        lax.fori_loop(0, NCH, chunk, 0)

    return k(idx_arr, table, pe_s)


def kernel(x, table, pe):
    idx_arr = (x.reshape(B, NW, NCH, R)
                .transpose(1, 2, 0, 3)
                .reshape(NW, NCH * B, R))
    out = _sc_embed(idx_arr, table, pe[:S])
    return out.reshape(B, S, D)
